# baseline (speedup 1.0000x reference)
import jax
import jax.numpy as jnp
from jax import lax
from jax.experimental import pallas as pl
from jax.experimental.pallas import tpu as pltpu
from jax.experimental.pallas import tpu_sc as plsc

_CHUNK_ROWS = 64


def kernel(x, enc_weight):
    seq_len = x.shape[1]
    d = enc_weight.shape[1]
    mesh = plsc.VectorSubcoreMesh(core_axis_name="c", subcore_axis_name="s")
    num_workers = mesh.num_cores * mesh.num_subcores
    rows_per_worker = seq_len // num_workers
    chunks = rows_per_worker // _CHUNK_ROWS

    def body(w_hbm, o_hbm, rows_v):
        wid = lax.axis_index("s") * mesh.num_cores + lax.axis_index("c")
        base = wid * rows_per_worker

        @pl.loop(0, chunks)
        def _(i):
            start = base + i * _CHUNK_ROWS
            pltpu.sync_copy(w_hbm.at[pl.ds(start, _CHUNK_ROWS)], rows_v)
            pltpu.sync_copy(rows_v, o_hbm.at[pl.ds(start, _CHUNK_ROWS)])

    return pl.kernel(
        body,
        out_type=jax.ShapeDtypeStruct((seq_len, d), enc_weight.dtype),
        mesh=mesh,
        scratch_types=[pltpu.VMEM((_CHUNK_ROWS, d), enc_weight.dtype)],
    )(enc_weight)
